# Initial kernel scaffold; baseline (speedup 1.0000x reference)
import jax
import jax.numpy as jnp
from jax.experimental import pallas as pl


def kernel(memory, last_update, nodes, values, ts):
    raise NotImplementedError("write your pallas kernel here")


# speedup vs baseline: 24.9458x; 24.9458x over previous
import functools

import jax
import jax.numpy as jnp
from jax import lax
from jax.experimental import pallas as pl
from jax.experimental.pallas import tpu as pltpu
from jax.experimental.pallas import tpu_sc as plsc

B = 16384
D = 128
NNODES = 1_000_000
NS = 16
L = 16
RANGE = 62504
CHUNK = B // NS
SUB = 128
NSUB = CHUNK // SUB
P1C = 1024


def _body(nodes_hbm, values_hbm, ts_hbm, mem_out_hbm, lu_out_hbm,
          nodes_buf, win_v, j_v, lu_v, rows_v, win_hbm):
    sid = lax.axis_index("s")

    base = sid * RANGE
    iota = lax.broadcasted_iota(jnp.int32, (L,), 0)

    for k in range(B // P1C):
        pltpu.sync_copy(nodes_hbm.at[pl.ds(k * P1C, P1C)], nodes_buf)

        def p1(t, carry, _k=k):
            n = nodes_buf[pl.ds(t * L, L)]
            key = n * L + iota
            skey, j_s = plsc.sort_key_val(key, _k * P1C + t * L + iota)
            n_s = skey >> 4
            nxt = n_s.at[jnp.minimum(iota + 1, L - 1)].get(
                mode="promise_in_bounds")
            loser = (n_s == nxt) & (iota < L - 1)
            m = (n_s >= base) & (n_s < base + RANGE) & jnp.logical_not(loser)
            plsc.store_scatter(win_v, [n_s - base], j_s, mask=m)
            return carry

        lax.fori_loop(0, P1C // L, p1, 0)

    pltpu.sync_copy(win_v, win_hbm.at[pl.ds(base, RANGE)])
    plsc.subcore_barrier()

    pltpu.sync_copy(nodes_hbm.at[pl.ds(sid * CHUNK, CHUNK)], nodes_buf)
    for c in range(NSUB):
        off = sid * CHUNK + c * SUB
        idx_ref = nodes_buf.at[pl.ds(c * SUB, SUB)]
        pltpu.sync_copy(win_hbm.at[idx_ref], j_v)
        pltpu.sync_copy(ts_hbm.at[j_v], lu_v)
        pltpu.sync_copy(values_hbm.at[j_v], rows_v)
        pltpu.sync_copy(rows_v, mem_out_hbm.at[pl.ds(off, SUB)])
        pltpu.sync_copy(lu_v, lu_out_hbm.at[pl.ds(off, SUB)])


_dedup_gather = functools.partial(
    pl.kernel,
    out_type=(
        jax.ShapeDtypeStruct((B, D), jnp.float32),
        jax.ShapeDtypeStruct((B,), jnp.float32),
    ),
    mesh=plsc.VectorSubcoreMesh(core_axis_name="c", subcore_axis_name="s",
                                num_cores=1),
    compiler_params=pltpu.CompilerParams(needs_layout_passes=False),
    scratch_types=[
        pltpu.VMEM((P1C,), jnp.int32),
        pltpu.VMEM((RANGE,), jnp.int32),
        pltpu.VMEM((SUB,), jnp.int32),
        pltpu.VMEM((SUB,), jnp.float32),
        pltpu.VMEM((SUB, D), jnp.float32),
        pltpu.HBM((NS * RANGE,), jnp.int32),
    ],
)(_body)


def kernel(memory, last_update, nodes, values, ts):
    mem_out, lu_out = _dedup_gather(nodes, values, ts)
    return mem_out, lu_out


# speedup vs baseline: 37.0359x; 1.4847x over previous
import functools

import jax
import jax.numpy as jnp
from jax import lax
from jax.experimental import pallas as pl
from jax.experimental.pallas import tpu as pltpu
from jax.experimental.pallas import tpu_sc as plsc

B = 16384
D = 128
NNODES = 1_000_000
NS = 16
L = 16
RANGE = 62504
CHUNK = B // NS
SUB = 128
NSUB = CHUNK // SUB
P1C = 1024
NP1 = B // P1C


def _body(nodes_hbm, values_hbm, ts_hbm, mem_out_hbm, lu_out_hbm,
          nodes_a, nodes_b, win_v, j_all, lu_all, rows_a, rows_b,
          sem_n, sem_j, sem_ts, sem_ga, sem_gb, sem_wa, sem_wb,
          win_hbm_ref):
    sid = lax.axis_index("s")

    base = sid * RANGE
    iota = lax.broadcasted_iota(jnp.int32, (L,), 0)
    nxt_idx = jnp.minimum(iota + 1, L - 1)
    nbufs = (nodes_a, nodes_b)

    cp = pltpu.async_copy(nodes_hbm.at[pl.ds(0, P1C)], nodes_a, sem_n)
    for k in range(NP1):
        cp.wait()
        if k + 1 < NP1:
            cp = pltpu.async_copy(nodes_hbm.at[pl.ds((k + 1) * P1C, P1C)],
                                  nbufs[(k + 1) % 2], sem_n)
        nbuf = nbufs[k % 2]

        def p1(t, carry, _k=k, _nbuf=nbuf):
            n = _nbuf[pl.ds(t * L, L)]
            key = n * L + iota
            skey, _ = plsc.sort_key_val(key, key)
            n_s = skey >> 4
            j_s = _k * P1C + t * L + (skey & (L - 1))
            nxt = n_s.at[nxt_idx].get(mode="promise_in_bounds")
            loser = (n_s == nxt) & (iota < L - 1)
            m = (n_s >= base) & (n_s < base + RANGE) & jnp.logical_not(loser)
            plsc.store_scatter(win_v, [n_s - base], j_s, mask=m)
            return carry

        lax.fori_loop(0, P1C // L, p1, 0)

    pltpu.sync_copy(win_v, win_hbm_ref.at[pl.ds(base, RANGE)])
    plsc.subcore_barrier()

    my = sid * CHUNK
    pltpu.sync_copy(nodes_hbm.at[pl.ds(my, CHUNK)], nodes_a)

    jcps = [pltpu.async_copy(win_hbm_ref.at[nodes_a.at[pl.ds(c * SUB, SUB)]],
                             j_all.at[pl.ds(c * SUB, SUB)], sem_j)
            for c in range(NSUB)]
    for c in jcps:
        c.wait()

    tcps = [pltpu.async_copy(ts_hbm.at[j_all.at[pl.ds(c * SUB, SUB)]],
                             lu_all.at[pl.ds(c * SUB, SUB)], sem_ts)
            for c in range(NSUB)]

    rbufs = (rows_a, rows_b)
    gsems = (sem_ga, sem_gb)
    wsems = (sem_wa, sem_wb)
    gcp = [None, None]
    wcp = [None, None]
    for c in range(2):
        gcp[c] = pltpu.async_copy(
            values_hbm.at[j_all.at[pl.ds(c * SUB, SUB)]], rbufs[c], gsems[c])
    for c in range(NSUB):
        b = c % 2
        gcp[b].wait()
        wcp[b] = pltpu.async_copy(
            rbufs[b], mem_out_hbm.at[pl.ds(my + c * SUB, SUB)], wsems[b])
        if c + 2 < NSUB:
            wcp[b].wait()
            gcp[b] = pltpu.async_copy(
                values_hbm.at[j_all.at[pl.ds((c + 2) * SUB, SUB)]],
                rbufs[b], gsems[b])
    wcp[0].wait()
    wcp[1].wait()
    for c in tcps:
        c.wait()
    pltpu.sync_copy(lu_all, lu_out_hbm.at[pl.ds(my, CHUNK)])


_dedup_gather = functools.partial(
    pl.kernel,
    out_type=(
        jax.ShapeDtypeStruct((B, D), jnp.float32),
        jax.ShapeDtypeStruct((B,), jnp.float32),
    ),
    mesh=plsc.VectorSubcoreMesh(core_axis_name="c", subcore_axis_name="s",
                                num_cores=1),
    compiler_params=pltpu.CompilerParams(needs_layout_passes=False),
    scratch_types=[
        pltpu.VMEM((P1C,), jnp.int32),
        pltpu.VMEM((P1C,), jnp.int32),
        pltpu.VMEM((RANGE,), jnp.int32),
        pltpu.VMEM((CHUNK,), jnp.int32),
        pltpu.VMEM((CHUNK,), jnp.float32),
        pltpu.VMEM((SUB, D), jnp.float32),
        pltpu.VMEM((SUB, D), jnp.float32),
        pltpu.SemaphoreType.DMA,
        pltpu.SemaphoreType.DMA,
        pltpu.SemaphoreType.DMA,
        pltpu.SemaphoreType.DMA,
        pltpu.SemaphoreType.DMA,
        pltpu.SemaphoreType.DMA,
        pltpu.SemaphoreType.DMA,
        pltpu.HBM((NS * RANGE,), jnp.int32),
    ],
)(_body)


def kernel(memory, last_update, nodes, values, ts):
    mem_out, lu_out = _dedup_gather(nodes, values, ts)
    return mem_out, lu_out
